# baseline (device time: 463819 ns/iter reference)
import jax
import jax.numpy as jnp
from jax import lax
from jax.experimental import pallas as pl
from jax.experimental.pallas import tpu as pltpu

N_DEV = 4
S = 2048
H = 8
DH = 128
D = 1024
QB = 128
NT = S // QB
SCALE = 0.08838834764831843
BLK = 64
NEG = -1e9
FIX_MAX = 10.0


def _dot(a, b, contract=((1,), (0,))):
    return lax.dot_general(
        a, b, (contract, ((), ())), preferred_element_type=jnp.float32
    )


def _body(x_ref, wq_ref, k_hbm, v_hbm, wo_ref, out_ref,
          ck, cv, mask_ref, acc_ref, l_ref,
          k_send, k_recv, v_send, v_recv, cp_sem):
    my = lax.axis_index("i")
    right = (my + 1) % N_DEV
    left = (my + N_DEV - 1) % N_DEV

    barrier = pltpu.get_barrier_semaphore()
    for nbr in (left, right):
        pl.semaphore_signal(barrier, inc=1, device_id=(nbr,),
                            device_id_type=pl.DeviceIdType.MESH)
    pl.semaphore_wait(barrier, 2)

    HH = H // 2

    def _hop(src_slot, dst_slot, hop):
        rs = []
        for buf, src_hbm, send, recv in (
                (ck, k_hbm, k_send, k_recv), (cv, v_hbm, v_send, v_recv)):
            for d, (lo, hi) in enumerate(((0, HH), (HH, H))):
                hsl = slice(lo, hi)
                src = (src_hbm.at[hsl] if src_slot is None
                       else buf.at[src_slot, hsl])
                r = pltpu.make_async_remote_copy(
                    src_ref=src, dst_ref=buf.at[dst_slot, hsl],
                    send_sem=send.at[d, hop], recv_sem=recv.at[d, hop],
                    device_id=(right,) if d == 0 else (left,),
                    device_id_type=pl.DeviceIdType.MESH)
                r.start()
                rs.append(r)
        return rs

    cp_k = pltpu.make_async_copy(k_hbm, ck.at[0], cp_sem.at[0])
    cp_v = pltpu.make_async_copy(v_hbm, cv.at[0], cp_sem.at[1])
    cp_k.start()
    cp_v.start()
    hop_rs = _hop(None, 1, 0)

    NR = QB // BLK
    qi = (lax.broadcasted_iota(jnp.int32, (NT, NR, S), 0) * NR
          + lax.broadcasted_iota(jnp.int32, (NT, NR, S), 1))
    kj = lax.broadcasted_iota(jnp.int32, (NT, NR, S), 2) // BLK
    mask_ref[...] = jnp.where(
        kj <= qi, jnp.float32(0.0), jnp.float32(NEG)
    ).astype(jnp.bfloat16)

    cp_k.wait()
    cp_v.wait()

    for s in range(N_DEV):
        if s > 0:
            for r in hop_rs:
                r.wait()
            if s < N_DEV - 1:
                hop_rs = _hop(s, s + 1, s)

        def q_tile(qb, carry, s=s):
            sl = pl.ds(qb * QB, QB)
            xq = x_ref[sl, :]
            mb = mask_ref[qb]

            def h_body(h, c):
                qh = (_dot(xq, wq_ref[h]) * SCALE).astype(jnp.bfloat16)
                sc = _dot(qh, ck[s, h], contract=((1,), (1,)))
                if s == 0:
                    sc3 = sc.reshape(QB // BLK, BLK, S)
                    sc = (sc3 + mb[:, None, :]).reshape(QB, S)
                else:
                    masked = jnp.where(h < HH, my < s, my < N_DEV - s)
                    sc = sc + jnp.where(masked, jnp.float32(NEG),
                                        jnp.float32(0.0))
                p = jnp.exp(sc - FIX_MAX)
                ls = jnp.sum(p, axis=1)
                pv = _dot(p.astype(jnp.bfloat16), cv[s, h])
                if s == 0:
                    l_ref[qb, h] = ls
                    acc_ref[h, sl, :] = pv.astype(jnp.bfloat16)
                else:
                    l_ref[qb, h] = l_ref[qb, h] + ls
                    acc_ref[h, sl, :] = (acc_ref[h, sl, :] + pv).astype(
                        jnp.bfloat16)
                return c

            lax.fori_loop(0, H, h_body, 0)
            return carry

        lax.fori_loop(0, NT, q_tile, 0)

    def fin_tile(qb, carry):
        sl = pl.ds(qb * QB, QB)

        def fh(h, o_acc):
            li = l_ref[qb, h]
            ctx = (acc_ref[h, sl, :] / li[:, None]).astype(jnp.bfloat16)
            return o_acc + _dot(ctx, wo_ref[h])

        o_acc = lax.fori_loop(0, H, fh, jnp.zeros((QB, D), jnp.float32))
        out_ref[sl, :] = o_acc.astype(jnp.bfloat16)
        return carry

    lax.fori_loop(0, NT, fin_tile, 0)


def kernel(x, Wq, K_ext, V_ext, Wo):
    xb = x[0].astype(jnp.bfloat16)
    wq = Wq.astype(jnp.bfloat16).reshape(D, H, DH).transpose(1, 0, 2)
    kt = K_ext[0].astype(jnp.bfloat16).transpose(1, 0, 2)
    vt = V_ext[0].astype(jnp.bfloat16).transpose(1, 0, 2)
    wo = Wo.astype(jnp.bfloat16).reshape(H, DH, D)

    out = pl.pallas_call(
        _body,
        out_shape=jax.ShapeDtypeStruct((S, D), jnp.bfloat16),
        in_specs=[
            pl.BlockSpec(memory_space=pltpu.VMEM),
            pl.BlockSpec(memory_space=pltpu.VMEM),
            pl.BlockSpec(memory_space=pl.ANY),
            pl.BlockSpec(memory_space=pl.ANY),
            pl.BlockSpec(memory_space=pltpu.VMEM),
        ],
        out_specs=pl.BlockSpec(memory_space=pltpu.VMEM),
        scratch_shapes=[
            pltpu.VMEM((N_DEV, H, S, DH), jnp.bfloat16),
            pltpu.VMEM((N_DEV, H, S, DH), jnp.bfloat16),
            pltpu.VMEM((NT, QB // BLK, S), jnp.bfloat16),
            pltpu.VMEM((H, S, DH), jnp.bfloat16),
            pltpu.VMEM((NT, H, QB), jnp.float32),
            pltpu.SemaphoreType.DMA((2, N_DEV - 1)),
            pltpu.SemaphoreType.DMA((2, N_DEV - 1)),
            pltpu.SemaphoreType.DMA((2, N_DEV - 1)),
            pltpu.SemaphoreType.DMA((2, N_DEV - 1)),
            pltpu.SemaphoreType.DMA((2,)),
        ],
        compiler_params=pltpu.CompilerParams(
            collective_id=0, vmem_limit_bytes=50 * 1024 * 1024
        ),
    )(xb, wq, kt, vt, wo)

    return out.astype(jnp.float32).reshape(1, S, D)


# device time: 372423 ns/iter; 1.2454x vs baseline; 1.2454x over previous
import jax
import jax.numpy as jnp
from jax import lax
from jax.experimental import pallas as pl
from jax.experimental.pallas import tpu as pltpu

N_DEV = 4
S = 2048
H = 8
DH = 128
D = 1024
QB = 128
NT = S // QB
SCALE = 0.08838834764831843
BLK = 64
NEG = -1e9
FIX_MAX = 10.0


def _dot(a, b, contract=((1,), (0,))):
    return lax.dot_general(
        a, b, (contract, ((), ())), preferred_element_type=jnp.float32
    )


def _body(x_ref, wq_ref, k_hbm, v_hbm, wo_ref, out_ref,
          ck, cv, mask_ref, acc_ref, l_ref,
          k_send, k_recv, v_send, v_recv, cp_sem):
    my = lax.axis_index("i")
    right = (my + 1) % N_DEV
    left = (my + N_DEV - 1) % N_DEV

    barrier = pltpu.get_barrier_semaphore()
    for nbr in (left, right):
        pl.semaphore_signal(barrier, inc=1, device_id=(nbr,),
                            device_id_type=pl.DeviceIdType.MESH)
    pl.semaphore_wait(barrier, 2)

    HH = H // 2

    def _hop(src_slot, dst_slot, hop):
        rs = []
        for buf, src_hbm, send, recv in (
                (ck, k_hbm, k_send, k_recv), (cv, v_hbm, v_send, v_recv)):
            for d, (lo, hi) in enumerate(((0, HH), (HH, H))):
                hsl = slice(lo, hi)
                src = (src_hbm.at[hsl] if src_slot is None
                       else buf.at[src_slot, hsl])
                r = pltpu.make_async_remote_copy(
                    src_ref=src, dst_ref=buf.at[dst_slot, hsl],
                    send_sem=send.at[d, hop], recv_sem=recv.at[d, hop],
                    device_id=(right,) if d == 0 else (left,),
                    device_id_type=pl.DeviceIdType.MESH)
                r.start()
                rs.append(r)
        return rs

    cp_k = pltpu.make_async_copy(k_hbm, ck.at[0], cp_sem.at[0])
    cp_v = pltpu.make_async_copy(v_hbm, cv.at[0], cp_sem.at[1])
    cp_k.start()
    cp_v.start()
    hop_rs = _hop(None, 1, 0)

    NR = QB // BLK
    qi = (lax.broadcasted_iota(jnp.int32, (NT, NR, S), 0) * NR
          + lax.broadcasted_iota(jnp.int32, (NT, NR, S), 1))
    kj = lax.broadcasted_iota(jnp.int32, (NT, NR, S), 2) // BLK
    mask_ref[...] = jnp.where(
        kj <= qi, jnp.float32(-FIX_MAX), jnp.float32(NEG)
    ).astype(jnp.bfloat16)

    cp_k.wait()
    cp_v.wait()

    for s in range(N_DEV):
        if s > 0:
            for r in hop_rs:
                r.wait()
            if s < N_DEV - 1:
                hop_rs = _hop(s, s + 1, s)
        last = s == N_DEV - 1

        def q_tile(qb, o_carry, s=s, last=last):
            sl = pl.ds(qb * QB, QB)
            xq = x_ref[sl, :]
            mb = mask_ref[qb]

            def one_head(h):
                qh = (_dot(xq, wq_ref[h]) * SCALE).astype(jnp.bfloat16)
                sc = _dot(qh, ck[s, h], contract=((1,), (1,)))
                if s == 0:
                    sc3 = sc.reshape(QB // BLK, BLK, S)
                    sc = (sc3 + mb[:, None, :]).reshape(QB, S)
                else:
                    masked = jnp.where(h < HH, my < s, my < N_DEV - s)
                    sc = sc + jnp.where(masked, jnp.float32(NEG),
                                        jnp.float32(-FIX_MAX))
                p = jnp.exp(sc)
                ls = jnp.sum(p, axis=1)
                pv = _dot(p.astype(jnp.bfloat16), cv[s, h])
                return ls, pv

            def h_body(hi, o_acc):
                res = [one_head(hi * 2), one_head(hi * 2 + 1)]
                for k, (ls, pv) in enumerate(res):
                    h = hi * 2 + k
                    if s == 0:
                        l_ref[qb, h] = ls
                        acc_ref[h, sl, :] = pv.astype(jnp.bfloat16)
                    elif not last:
                        l_ref[qb, h] = l_ref[qb, h] + ls
                        acc_ref[h, sl, :] = (acc_ref[h, sl, :] + pv).astype(
                            jnp.bfloat16)
                    else:
                        lt = l_ref[qb, h] + ls
                        at = acc_ref[h, sl, :] + pv
                        ctx = (at / lt[:, None]).astype(jnp.bfloat16)
                        o_acc = o_acc + _dot(ctx, wo_ref[h])
                return o_acc

            o_acc = lax.fori_loop(
                0, H // 2, h_body, jnp.zeros((QB, D), jnp.float32))
            if last:
                out_ref[sl, :] = o_acc.astype(jnp.bfloat16)
            return o_carry

        lax.fori_loop(0, NT, q_tile, 0)


def kernel(x, Wq, K_ext, V_ext, Wo):
    xb = x[0].astype(jnp.bfloat16)
    wq = Wq.astype(jnp.bfloat16).reshape(D, H, DH).transpose(1, 0, 2)
    kt = K_ext[0].astype(jnp.bfloat16).transpose(1, 0, 2)
    vt = V_ext[0].astype(jnp.bfloat16).transpose(1, 0, 2)
    wo = Wo.astype(jnp.bfloat16).reshape(H, DH, D)

    out = pl.pallas_call(
        _body,
        out_shape=jax.ShapeDtypeStruct((S, D), jnp.bfloat16),
        in_specs=[
            pl.BlockSpec(memory_space=pltpu.VMEM),
            pl.BlockSpec(memory_space=pltpu.VMEM),
            pl.BlockSpec(memory_space=pl.ANY),
            pl.BlockSpec(memory_space=pl.ANY),
            pl.BlockSpec(memory_space=pltpu.VMEM),
        ],
        out_specs=pl.BlockSpec(memory_space=pltpu.VMEM),
        scratch_shapes=[
            pltpu.VMEM((N_DEV, H, S, DH), jnp.bfloat16),
            pltpu.VMEM((N_DEV, H, S, DH), jnp.bfloat16),
            pltpu.VMEM((NT, QB // BLK, S), jnp.bfloat16),
            pltpu.VMEM((H, S, DH), jnp.bfloat16),
            pltpu.VMEM((NT, H, QB), jnp.float32),
            pltpu.SemaphoreType.DMA((2, N_DEV - 1)),
            pltpu.SemaphoreType.DMA((2, N_DEV - 1)),
            pltpu.SemaphoreType.DMA((2, N_DEV - 1)),
            pltpu.SemaphoreType.DMA((2, N_DEV - 1)),
            pltpu.SemaphoreType.DMA((2,)),
        ],
        compiler_params=pltpu.CompilerParams(
            collective_id=0, vmem_limit_bytes=50 * 1024 * 1024
        ),
    )(xb, wq, kt, vt, wo)

    return out.astype(jnp.float32).reshape(1, S, D)


# device time: 342698 ns/iter; 1.3534x vs baseline; 1.0867x over previous
import jax
import jax.numpy as jnp
from jax import lax
from jax.experimental import pallas as pl
from jax.experimental.pallas import tpu as pltpu

N_DEV = 4
S = 2048
H = 8
DH = 128
D = 1024
QB = 128
NT = S // QB
SCALE = 0.08838834764831843
BLK = 64
NEG = -1e9
FIX_MAX = 10.0


def _dot(a, b, contract=((1,), (0,))):
    return lax.dot_general(
        a, b, (contract, ((), ())), preferred_element_type=jnp.float32
    )


def _body(x_ref, wq_ref, k_hbm, v_hbm, wo_ref, out_ref,
          ck, cv, mask_ref, acc_ref, l_ref,
          k_send, k_recv, v_send, v_recv, cp_sem):
    my = lax.axis_index("i")
    right = (my + 1) % N_DEV
    left = (my + N_DEV - 1) % N_DEV

    barrier = pltpu.get_barrier_semaphore()
    for nbr in (left, right):
        pl.semaphore_signal(barrier, inc=1, device_id=(nbr,),
                            device_id_type=pl.DeviceIdType.MESH)
    pl.semaphore_wait(barrier, 2)

    HH = H // 2

    def _hop(src_slot, dst_slot, hop):
        rs = []
        for buf, src_hbm, send, recv in (
                (ck, k_hbm, k_send, k_recv), (cv, v_hbm, v_send, v_recv)):
            for d, (lo, hi) in enumerate(((0, HH), (HH, H))):
                hsl = slice(lo, hi)
                src = (src_hbm.at[hsl] if src_slot is None
                       else buf.at[src_slot, hsl])
                r = pltpu.make_async_remote_copy(
                    src_ref=src, dst_ref=buf.at[dst_slot, hsl],
                    send_sem=send.at[d, hop], recv_sem=recv.at[d, hop],
                    device_id=(right,) if d == 0 else (left,),
                    device_id_type=pl.DeviceIdType.MESH)
                r.start()
                rs.append(r)
        return rs

    cp_k = pltpu.make_async_copy(k_hbm, ck.at[0], cp_sem.at[0])
    cp_v = pltpu.make_async_copy(v_hbm, cv.at[0], cp_sem.at[1])
    cp_k.start()
    cp_v.start()
    hop_rs = _hop(None, 1, 0)

    NR = QB // BLK
    qi = (lax.broadcasted_iota(jnp.int32, (NT, NR, S), 0) * NR
          + lax.broadcasted_iota(jnp.int32, (NT, NR, S), 1))
    kj = lax.broadcasted_iota(jnp.int32, (NT, NR, S), 2) // BLK
    mask_ref[...] = jnp.where(
        kj <= qi, jnp.float32(-FIX_MAX), jnp.float32(NEG)
    ).astype(jnp.bfloat16)

    cp_k.wait()
    cp_v.wait()

    for s in range(N_DEV):
        if s > 0:
            for r in hop_rs:
                r.wait()
            if s < N_DEV - 1:
                hop_rs = _hop(s, s + 1, s)
        last = s == N_DEV - 1

        def q_tile(qb, o_carry, s=s, last=last):
            sl = pl.ds(qb * QB, QB)
            xq = x_ref[sl, :]
            mb = mask_ref[qb]

            def one_head(h):
                qh = (_dot(xq, wq_ref[h]) * SCALE).astype(jnp.bfloat16)
                sc = _dot(qh, ck[s, h], contract=((1,), (1,)))
                if s == 0:
                    sc3 = sc.reshape(QB // BLK, BLK, S)
                    sc = (sc3 + mb[:, None, :]).reshape(QB, S)
                else:
                    masked = jnp.where(h < HH, my < s, my < N_DEV - s)
                    sc = sc + jnp.where(masked, jnp.float32(NEG),
                                        jnp.float32(-FIX_MAX))
                p = jnp.exp(sc)
                ls = jnp.sum(p, axis=1)
                pv = _dot(p.astype(jnp.bfloat16), cv[s, h])
                return ls, pv

            def h_body(hi, o_acc):
                res = [one_head(hi * 4 + k) for k in range(4)]
                for k, (ls, pv) in enumerate(res):
                    h = hi * 4 + k
                    if s == 0:
                        l_ref[qb, h] = ls
                        acc_ref[h, sl, :] = pv.astype(jnp.bfloat16)
                    elif not last:
                        l_ref[qb, h] = l_ref[qb, h] + ls
                        acc_ref[h, sl, :] = (acc_ref[h, sl, :] + pv).astype(
                            jnp.bfloat16)
                    else:
                        lt = l_ref[qb, h] + ls
                        at = acc_ref[h, sl, :] + pv
                        ctx = (at / lt[:, None]).astype(jnp.bfloat16)
                        o_acc = o_acc + _dot(ctx, wo_ref[h])
                return o_acc

            o_acc = lax.fori_loop(
                0, H // 4, h_body, jnp.zeros((QB, D), jnp.float32))
            if last:
                out_ref[sl, :] = o_acc.astype(jnp.bfloat16)
            return o_carry

        lax.fori_loop(0, NT, q_tile, 0)


def kernel(x, Wq, K_ext, V_ext, Wo):
    xb = x[0].astype(jnp.bfloat16)
    wq = Wq.astype(jnp.bfloat16).reshape(D, H, DH).transpose(1, 0, 2)
    kt = K_ext[0].astype(jnp.bfloat16).transpose(1, 0, 2)
    vt = V_ext[0].astype(jnp.bfloat16).transpose(1, 0, 2)
    wo = Wo.astype(jnp.bfloat16).reshape(H, DH, D)

    out = pl.pallas_call(
        _body,
        out_shape=jax.ShapeDtypeStruct((S, D), jnp.bfloat16),
        in_specs=[
            pl.BlockSpec(memory_space=pltpu.VMEM),
            pl.BlockSpec(memory_space=pltpu.VMEM),
            pl.BlockSpec(memory_space=pl.ANY),
            pl.BlockSpec(memory_space=pl.ANY),
            pl.BlockSpec(memory_space=pltpu.VMEM),
        ],
        out_specs=pl.BlockSpec(memory_space=pltpu.VMEM),
        scratch_shapes=[
            pltpu.VMEM((N_DEV, H, S, DH), jnp.bfloat16),
            pltpu.VMEM((N_DEV, H, S, DH), jnp.bfloat16),
            pltpu.VMEM((NT, QB // BLK, S), jnp.bfloat16),
            pltpu.VMEM((H, S, DH), jnp.bfloat16),
            pltpu.VMEM((NT, H, QB), jnp.float32),
            pltpu.SemaphoreType.DMA((2, N_DEV - 1)),
            pltpu.SemaphoreType.DMA((2, N_DEV - 1)),
            pltpu.SemaphoreType.DMA((2, N_DEV - 1)),
            pltpu.SemaphoreType.DMA((2, N_DEV - 1)),
            pltpu.SemaphoreType.DMA((2,)),
        ],
        compiler_params=pltpu.CompilerParams(
            collective_id=0, vmem_limit_bytes=50 * 1024 * 1024
        ),
    )(xb, wq, kt, vt, wo)

    return out.astype(jnp.float32).reshape(1, S, D)


# device time: 252944 ns/iter; 1.8337x vs baseline; 1.3548x over previous
import jax
import jax.numpy as jnp
from jax import lax
from jax.experimental import pallas as pl
from jax.experimental.pallas import tpu as pltpu

N_DEV = 4
S = 2048
H = 8
DH = 128
D = 1024
QB = 128
NT = S // QB
SCALE = 0.08838834764831843
BLK = 64
NEG = -1e9
FIX_MAX = 10.0


def _dot(a, b, contract=((1,), (0,))):
    return lax.dot_general(
        a, b, (contract, ((), ())), preferred_element_type=jnp.float32
    )


def _body(x_ref, wq_ref, k_hbm, v_hbm, wo_ref, out_ref,
          ck, cv, mask_ref, acc_ref, l_ref, q_ref, stage_ref,
          k_send, k_recv, v_send, v_recv, cp_sem, out_sem):
    my = lax.axis_index("i")
    right = (my + 1) % N_DEV
    left = (my + N_DEV - 1) % N_DEV

    barrier = pltpu.get_barrier_semaphore()
    for nbr in (left, right):
        pl.semaphore_signal(barrier, inc=1, device_id=(nbr,),
                            device_id_type=pl.DeviceIdType.MESH)
    pl.semaphore_wait(barrier, 2)

    HH = H // 2

    def _hop(src_slot, dst_slot, hop):
        rs = []
        for buf, src_hbm, send, recv in (
                (ck, k_hbm, k_send, k_recv), (cv, v_hbm, v_send, v_recv)):
            for d, (lo, hi) in enumerate(((0, HH), (HH, H))):
                hsl = slice(lo, hi)
                src = (src_hbm.at[hsl] if src_slot is None
                       else buf.at[src_slot, hsl])
                r = pltpu.make_async_remote_copy(
                    src_ref=src, dst_ref=buf.at[dst_slot, hsl],
                    send_sem=send.at[d, hop], recv_sem=recv.at[d, hop],
                    device_id=(right,) if d == 0 else (left,),
                    device_id_type=pl.DeviceIdType.MESH)
                r.start()
                rs.append(r)
        return rs

    cp_k = pltpu.make_async_copy(k_hbm, ck.at[0], cp_sem.at[0])
    cp_v = pltpu.make_async_copy(v_hbm, cv.at[0], cp_sem.at[1])
    cp_k.start()
    cp_v.start()
    hop_rs = _hop(None, 1, 0)

    NR = QB // BLK
    qi = (lax.broadcasted_iota(jnp.int32, (NT, NR, S), 0) * NR
          + lax.broadcasted_iota(jnp.int32, (NT, NR, S), 1))
    kj = lax.broadcasted_iota(jnp.int32, (NT, NR, S), 2) // BLK
    mask_ref[...] = jnp.where(
        kj <= qi, jnp.float32(0.0), jnp.float32(NEG)
    ).astype(jnp.bfloat16)

    def q_pre(h, c):
        q_ref[h] = (_dot(x_ref[...], wq_ref[h]) * SCALE).astype(jnp.bfloat16)
        return c

    lax.fori_loop(0, H, q_pre, 0)

    cp_k.wait()
    cp_v.wait()

    for s in range(N_DEV):
        if s > 0:
            for r in hop_rs:
                r.wait()
            if s < N_DEV - 1:
                hop_rs = _hop(s, s + 1, s)
        last = s == N_DEV - 1

        def q_tile(qb, o_carry, s=s, last=last):
            sl = pl.ds(qb * QB, QB)
            mb = mask_ref[qb]

            def one_head(h):
                qh = q_ref[h, sl, :]
                sc = _dot(qh, ck[s, h], contract=((1,), (1,)))
                if s == 0:
                    sc3 = sc.reshape(QB // BLK, BLK, S)
                    sc = (sc3 + mb[:, None, :]).reshape(QB, S)
                p = jnp.exp(sc)
                ls = jnp.sum(p, axis=1)
                pv = _dot(p.astype(jnp.bfloat16), cv[s, h])
                if s > 0:
                    masked = jnp.where(h < HH, my < s, my < N_DEV - s)
                    gate = jnp.where(masked, jnp.float32(0.0),
                                     jnp.float32(1.0))
                    ls = ls * gate
                    pv = pv * gate
                return ls, pv

            def h_body(hi, o_acc):
                res = [one_head(hi * 4 + k) for k in range(4)]
                for k, (ls, pv) in enumerate(res):
                    h = hi * 4 + k
                    if s == 0:
                        l_ref[qb, h] = ls
                        acc_ref[h, sl, :] = pv.astype(jnp.bfloat16)
                    elif not last:
                        l_ref[qb, h] = l_ref[qb, h] + ls
                        acc_ref[h, sl, :] = (acc_ref[h, sl, :] + pv).astype(
                            jnp.bfloat16)
                    else:
                        lt = l_ref[qb, h] + ls
                        at = acc_ref[h, sl, :] + pv
                        ctx = (at / lt[:, None]).astype(jnp.bfloat16)
                        o_acc = o_acc + _dot(ctx, wo_ref[h])
                return o_acc

            o_acc = lax.fori_loop(
                0, H // 4, h_body, jnp.zeros((QB, D), jnp.float32))
            if last:
                stage_ref[...] = o_acc.astype(jnp.bfloat16)
                cp = pltpu.make_async_copy(
                    stage_ref, out_ref.at[pl.ds(qb * QB, QB)], out_sem)
                cp.start()
                cp.wait()
            return o_carry

        lax.fori_loop(0, NT, q_tile, 0)


def kernel(x, Wq, K_ext, V_ext, Wo):
    xb = x[0].astype(jnp.bfloat16)
    wq = Wq.astype(jnp.bfloat16).reshape(D, H, DH).transpose(1, 0, 2)
    kt = K_ext[0].astype(jnp.bfloat16).transpose(1, 0, 2)
    vt = V_ext[0].astype(jnp.bfloat16).transpose(1, 0, 2)
    wo = Wo.astype(jnp.bfloat16).reshape(H, DH, D)

    out = pl.pallas_call(
        _body,
        out_shape=jax.ShapeDtypeStruct((S, D), jnp.bfloat16),
        in_specs=[
            pl.BlockSpec(memory_space=pltpu.VMEM),
            pl.BlockSpec(memory_space=pltpu.VMEM),
            pl.BlockSpec(memory_space=pl.ANY),
            pl.BlockSpec(memory_space=pl.ANY),
            pl.BlockSpec(memory_space=pltpu.VMEM),
        ],
        out_specs=pl.BlockSpec(memory_space=pl.ANY),
        scratch_shapes=[
            pltpu.VMEM((N_DEV, H, S, DH), jnp.bfloat16),
            pltpu.VMEM((N_DEV, H, S, DH), jnp.bfloat16),
            pltpu.VMEM((NT, QB // BLK, S), jnp.bfloat16),
            pltpu.VMEM((H, S, DH), jnp.bfloat16),
            pltpu.VMEM((NT, H, QB), jnp.float32),
            pltpu.VMEM((H, S, DH), jnp.bfloat16),
            pltpu.VMEM((QB, D), jnp.bfloat16),
            pltpu.SemaphoreType.DMA((2, N_DEV - 1)),
            pltpu.SemaphoreType.DMA((2, N_DEV - 1)),
            pltpu.SemaphoreType.DMA((2, N_DEV - 1)),
            pltpu.SemaphoreType.DMA((2, N_DEV - 1)),
            pltpu.SemaphoreType.DMA((2,)),
            pltpu.SemaphoreType.DMA,
        ],
        compiler_params=pltpu.CompilerParams(
            collective_id=0, vmem_limit_bytes=50 * 1024 * 1024
        ),
    )(xb, wq, kt, vt, wo)

    return out.astype(jnp.float32).reshape(1, S, D)


# device time: 242253 ns/iter; 1.9146x vs baseline; 1.0441x over previous
import jax
import jax.numpy as jnp
from jax import lax
from jax.experimental import pallas as pl
from jax.experimental.pallas import tpu as pltpu

N_DEV = 4
S = 2048
H = 8
DH = 128
D = 1024
QB = 128
NT = S // QB
SCALE = 0.08838834764831843
BLK = 64
NEG = -1e9
FIX_MAX = 10.0


def _dot(a, b, contract=((1,), (0,))):
    return lax.dot_general(
        a, b, (contract, ((), ())), preferred_element_type=jnp.float32
    )


def _body(x_ref, wq_ref, k_hbm, v_hbm, wo_ref, out_ref,
          ck, cv, mask_ref, acc_ref, l_ref, q_ref, stage_ref,
          k_send, k_recv, v_send, v_recv, cp_sem, out_sem):
    my = lax.axis_index("i")
    right = (my + 1) % N_DEV
    left = (my + N_DEV - 1) % N_DEV

    barrier = pltpu.get_barrier_semaphore()
    for nbr in (left, right):
        pl.semaphore_signal(barrier, inc=1, device_id=(nbr,),
                            device_id_type=pl.DeviceIdType.MESH)
    pl.semaphore_wait(barrier, 2)

    HH = H // 2

    def _hop(src_slot, dst_slot, hop):
        rs = []
        for buf, src_hbm, send, recv in (
                (ck, k_hbm, k_send, k_recv), (cv, v_hbm, v_send, v_recv)):
            for d, (lo, hi) in enumerate(((0, HH), (HH, H))):
                hsl = slice(lo, hi)
                src = (src_hbm.at[hsl] if src_slot is None
                       else buf.at[src_slot, hsl])
                r = pltpu.make_async_remote_copy(
                    src_ref=src, dst_ref=buf.at[dst_slot, hsl],
                    send_sem=send.at[d, hop], recv_sem=recv.at[d, hop],
                    device_id=(right,) if d == 0 else (left,),
                    device_id_type=pl.DeviceIdType.MESH)
                r.start()
                rs.append(r)
        return rs

    cp_k = pltpu.make_async_copy(k_hbm, ck.at[0], cp_sem.at[0])
    cp_v = pltpu.make_async_copy(v_hbm, cv.at[0], cp_sem.at[1])
    cp_k.start()
    cp_v.start()
    hop_rs = _hop(None, 1, 0)

    NR = QB // BLK
    qi = (lax.broadcasted_iota(jnp.int32, (NT, NR, S), 0) * NR
          + lax.broadcasted_iota(jnp.int32, (NT, NR, S), 1))
    kj = lax.broadcasted_iota(jnp.int32, (NT, NR, S), 2) // BLK
    mask_ref[...] = jnp.where(
        kj <= qi, jnp.float32(0.0), jnp.float32(NEG)
    ).astype(jnp.bfloat16)

    def q_pre(h, c):
        q_ref[h] = (_dot(x_ref[...], wq_ref[h]) * SCALE).astype(jnp.bfloat16)
        return c

    lax.fori_loop(0, H, q_pre, 0)

    cp_k.wait()
    cp_v.wait()

    for s in range(N_DEV):
        if s > 0:
            for r in hop_rs:
                r.wait()
            if s < N_DEV - 1:
                hop_rs = _hop(s, s + 1, s)
        last = s == N_DEV - 1

        def q_tile(qb, o_carry, s=s, last=last):
            sl = pl.ds(qb * QB, QB)
            mb = mask_ref[qb]

            def one_head(h):
                qh = q_ref[h, sl, :]
                sc = _dot(qh, ck[s, h], contract=((1,), (1,)))
                if s == 0:
                    sc3 = sc.reshape(QB // BLK, BLK, S)
                    sc = (sc3 + mb[:, None, :]).reshape(QB, S)
                p = jnp.exp(sc)
                ls = jnp.sum(p, axis=1)
                pv = _dot(p.astype(jnp.bfloat16), cv[s, h])
                if s > 0:
                    masked = jnp.where(h < HH, my < s, my < N_DEV - s)
                    gate = jnp.where(masked, jnp.float32(0.0),
                                     jnp.float32(1.0))
                    ls = ls * gate
                    pv = pv * gate
                return ls, pv

            def h_body(hi, o_acc):
                res = [one_head(hi * 8 + k) for k in range(8)]
                for k, (ls, pv) in enumerate(res):
                    h = hi * 8 + k
                    if s == 0:
                        l_ref[qb, h] = ls
                        acc_ref[h, sl, :] = pv.astype(jnp.bfloat16)
                    elif not last:
                        l_ref[qb, h] = l_ref[qb, h] + ls
                        acc_ref[h, sl, :] = (acc_ref[h, sl, :] + pv).astype(
                            jnp.bfloat16)
                    else:
                        lt = l_ref[qb, h] + ls
                        at = acc_ref[h, sl, :] + pv
                        ctx = (at / lt[:, None]).astype(jnp.bfloat16)
                        o_acc = o_acc + _dot(ctx, wo_ref[h])
                return o_acc

            o_acc = lax.fori_loop(
                0, H // 8, h_body, jnp.zeros((QB, D), jnp.float32))
            if last:
                stage_ref[...] = o_acc.astype(jnp.bfloat16)
                cp = pltpu.make_async_copy(
                    stage_ref, out_ref.at[pl.ds(qb * QB, QB)], out_sem)
                cp.start()
                cp.wait()
            return o_carry

        lax.fori_loop(0, NT, q_tile, 0)


def kernel(x, Wq, K_ext, V_ext, Wo):
    xb = x[0].astype(jnp.bfloat16)
    wq = Wq.astype(jnp.bfloat16).reshape(D, H, DH).transpose(1, 0, 2)
    kt = K_ext[0].astype(jnp.bfloat16).transpose(1, 0, 2)
    vt = V_ext[0].astype(jnp.bfloat16).transpose(1, 0, 2)
    wo = Wo.astype(jnp.bfloat16).reshape(H, DH, D)

    out = pl.pallas_call(
        _body,
        out_shape=jax.ShapeDtypeStruct((S, D), jnp.bfloat16),
        in_specs=[
            pl.BlockSpec(memory_space=pltpu.VMEM),
            pl.BlockSpec(memory_space=pltpu.VMEM),
            pl.BlockSpec(memory_space=pl.ANY),
            pl.BlockSpec(memory_space=pl.ANY),
            pl.BlockSpec(memory_space=pltpu.VMEM),
        ],
        out_specs=pl.BlockSpec(memory_space=pl.ANY),
        scratch_shapes=[
            pltpu.VMEM((N_DEV, H, S, DH), jnp.bfloat16),
            pltpu.VMEM((N_DEV, H, S, DH), jnp.bfloat16),
            pltpu.VMEM((NT, QB // BLK, S), jnp.bfloat16),
            pltpu.VMEM((H, S, DH), jnp.bfloat16),
            pltpu.VMEM((NT, H, QB), jnp.float32),
            pltpu.VMEM((H, S, DH), jnp.bfloat16),
            pltpu.VMEM((QB, D), jnp.bfloat16),
            pltpu.SemaphoreType.DMA((2, N_DEV - 1)),
            pltpu.SemaphoreType.DMA((2, N_DEV - 1)),
            pltpu.SemaphoreType.DMA((2, N_DEV - 1)),
            pltpu.SemaphoreType.DMA((2, N_DEV - 1)),
            pltpu.SemaphoreType.DMA((2,)),
            pltpu.SemaphoreType.DMA,
        ],
        compiler_params=pltpu.CompilerParams(
            collective_id=0, vmem_limit_bytes=50 * 1024 * 1024
        ),
    )(xb, wq, kt, vt, wo)

    return out.astype(jnp.float32).reshape(1, S, D)


# device time: 234482 ns/iter; 1.9781x vs baseline; 1.0331x over previous
import functools

import jax
import jax.numpy as jnp
from jax import lax
from jax.experimental import pallas as pl
from jax.experimental.pallas import tpu as pltpu

N_DEV = 4
S = 2048
H = 8
DH = 128
D = 1024
QB = 128
NT = S // QB
SCALE = 0.08838834764831843
BLK = 64
NEG = -1e9
FIX_MAX = 10.0


def _dot(a, b, contract=((1,), (0,))):
    return lax.dot_general(
        a, b, (contract, ((), ())), preferred_element_type=jnp.float32
    )


def _body(x_ref, wq_ref, k_hbm, v_hbm, wo_ref, out_ref,
          ck, cv, mask_ref, acc_ref, l_ref, q_ref, stage_ref,
          k_send, k_recv, v_send, v_recv, cp_sem, out_sem):
    my = lax.axis_index("i")
    right = (my + 1) % N_DEV
    left = (my + N_DEV - 1) % N_DEV

    barrier = pltpu.get_barrier_semaphore()
    for nbr in (left, right):
        pl.semaphore_signal(barrier, inc=1, device_id=(nbr,),
                            device_id_type=pl.DeviceIdType.MESH)
    pl.semaphore_wait(barrier, 2)

    HH = H // 2

    def _hop(src_slot, dst_slot, hop):
        rs = []
        for buf, src_hbm, send, recv in (
                (ck, k_hbm, k_send, k_recv), (cv, v_hbm, v_send, v_recv)):
            for d, (lo, hi) in enumerate(((0, HH), (HH, H))):
                hsl = slice(lo, hi)
                src = (src_hbm.at[hsl] if src_slot is None
                       else buf.at[src_slot, hsl])
                r = pltpu.make_async_remote_copy(
                    src_ref=src, dst_ref=buf.at[dst_slot, hsl],
                    send_sem=send.at[d, hop], recv_sem=recv.at[d, hop],
                    device_id=(right,) if d == 0 else (left,),
                    device_id_type=pl.DeviceIdType.MESH)
                r.start()
                rs.append(r)
        return rs

    cp_k = pltpu.make_async_copy(k_hbm, ck.at[0], cp_sem.at[0])
    cp_v = pltpu.make_async_copy(v_hbm, cv.at[0], cp_sem.at[1])
    cp_k.start()
    cp_v.start()
    hop_rs = _hop(None, 1, 0)

    NR = QB // BLK
    qi = (lax.broadcasted_iota(jnp.int32, (NT, NR, S), 0) * NR
          + lax.broadcasted_iota(jnp.int32, (NT, NR, S), 1))
    kj = lax.broadcasted_iota(jnp.int32, (NT, NR, S), 2) // BLK
    mask_ref[...] = jnp.where(
        kj <= qi, jnp.float32(0.0), jnp.float32(NEG)
    ).astype(jnp.bfloat16)

    def q_pre(h, c):
        q_ref[h] = (_dot(x_ref[...], wq_ref[h]) * SCALE).astype(jnp.bfloat16)
        return c

    lax.fori_loop(0, H, q_pre, 0)

    cp_k.wait()
    cp_v.wait()

    for s in range(N_DEV):
        if s > 0:
            for r in hop_rs:
                r.wait()
            if s < N_DEV - 1:
                hop_rs = _hop(s, s + 1, s)
        last = s == N_DEV - 1

        def q_tile(qb, o_carry, s=s, last=last, kvlen=S):
            sl = pl.ds(qb * QB, QB)
            mb = mask_ref[qb]

            def one_head(h):
                qh = q_ref[h, sl, :]
                sc = _dot(qh, ck[s, h, :kvlen],
                          contract=((1,), (1,)))
                if s == 0:
                    sc3 = sc.reshape(QB // BLK, BLK, kvlen)
                    sc = (sc3 + mb[:, None, :kvlen]).reshape(QB, kvlen)
                p = jnp.exp(sc)
                ls = jnp.sum(p, axis=1)
                pv = _dot(p.astype(jnp.bfloat16),
                          cv[s, h, :kvlen])
                if s > 0:
                    masked = jnp.where(h < HH, my < s, my < N_DEV - s)
                    gate = jnp.where(masked, jnp.float32(0.0),
                                     jnp.float32(1.0))
                    ls = ls * gate
                    pv = pv * gate
                return ls, pv

            def h_body(hi, o_acc):
                res = [one_head(hi * 8 + k) for k in range(8)]
                for k, (ls, pv) in enumerate(res):
                    h = hi * 8 + k
                    if s == 0:
                        l_ref[qb, h] = ls
                        acc_ref[h, sl, :] = pv.astype(jnp.bfloat16)
                    elif not last:
                        l_ref[qb, h] = l_ref[qb, h] + ls
                        acc_ref[h, sl, :] = (acc_ref[h, sl, :] + pv).astype(
                            jnp.bfloat16)
                    else:
                        lt = l_ref[qb, h] + ls
                        at = acc_ref[h, sl, :] + pv
                        ctx = (at / lt[:, None]).astype(jnp.bfloat16)
                        o_acc = o_acc + _dot(ctx, wo_ref[h])
                return o_acc

            o_acc = lax.fori_loop(
                0, H // 8, h_body, jnp.zeros((QB, D), jnp.float32))
            if last:
                stage_ref[...] = o_acc.astype(jnp.bfloat16)
                cp = pltpu.make_async_copy(
                    stage_ref, out_ref.at[pl.ds(qb * QB, QB)], out_sem)
                cp.start()
                cp.wait()
            return o_carry

        if s == 0:
            for g in range(4):
                lax.fori_loop(
                    g * (NT // 4), (g + 1) * (NT // 4),
                    functools.partial(q_tile, kvlen=(g + 1) * (S // 4)), 0)
        else:
            lax.fori_loop(0, NT, q_tile, 0)


def kernel(x, Wq, K_ext, V_ext, Wo):
    xb = x[0].astype(jnp.bfloat16)
    wq = Wq.astype(jnp.bfloat16).reshape(D, H, DH).transpose(1, 0, 2)
    kt = K_ext[0].astype(jnp.bfloat16).transpose(1, 0, 2)
    vt = V_ext[0].astype(jnp.bfloat16).transpose(1, 0, 2)
    wo = Wo.astype(jnp.bfloat16).reshape(H, DH, D)

    out = pl.pallas_call(
        _body,
        out_shape=jax.ShapeDtypeStruct((S, D), jnp.bfloat16),
        in_specs=[
            pl.BlockSpec(memory_space=pltpu.VMEM),
            pl.BlockSpec(memory_space=pltpu.VMEM),
            pl.BlockSpec(memory_space=pl.ANY),
            pl.BlockSpec(memory_space=pl.ANY),
            pl.BlockSpec(memory_space=pltpu.VMEM),
        ],
        out_specs=pl.BlockSpec(memory_space=pl.ANY),
        scratch_shapes=[
            pltpu.VMEM((N_DEV, H, S, DH), jnp.bfloat16),
            pltpu.VMEM((N_DEV, H, S, DH), jnp.bfloat16),
            pltpu.VMEM((NT, QB // BLK, S), jnp.bfloat16),
            pltpu.VMEM((H, S, DH), jnp.bfloat16),
            pltpu.VMEM((NT, H, QB), jnp.float32),
            pltpu.VMEM((H, S, DH), jnp.bfloat16),
            pltpu.VMEM((QB, D), jnp.bfloat16),
            pltpu.SemaphoreType.DMA((2, N_DEV - 1)),
            pltpu.SemaphoreType.DMA((2, N_DEV - 1)),
            pltpu.SemaphoreType.DMA((2, N_DEV - 1)),
            pltpu.SemaphoreType.DMA((2, N_DEV - 1)),
            pltpu.SemaphoreType.DMA((2,)),
            pltpu.SemaphoreType.DMA,
        ],
        compiler_params=pltpu.CompilerParams(
            collective_id=0, vmem_limit_bytes=50 * 1024 * 1024
        ),
    )(xb, wq, kt, vt, wo)

    return out.astype(jnp.float32).reshape(1, S, D)


# device time: 233930 ns/iter; 1.9827x vs baseline; 1.0024x over previous
import functools

import jax
import jax.numpy as jnp
from jax import lax
from jax.experimental import pallas as pl
from jax.experimental.pallas import tpu as pltpu

N_DEV = 4
S = 2048
H = 8
DH = 128
D = 1024
QB = 128
NT = S // QB
SCALE = 0.08838834764831843
BLK = 64
NEG = -1e9


def _dot(a, b, contract=((1,), (0,))):
    return lax.dot_general(
        a, b, (contract, ((), ())), preferred_element_type=jnp.float32
    )


def _body(x_ref, wq_ref, k_hbm, v_hbm, wo_ref, out_ref,
          ck, cv, mask_ref, acc_ref, l_ref, q_ref, stage_ref,
          k_send, k_recv, v_send, v_recv, cp_sem, out_sem):
    my = lax.axis_index("i")
    right = (my + 1) % N_DEV
    left = (my + N_DEV - 1) % N_DEV

    barrier = pltpu.get_barrier_semaphore()
    for nbr in (left, right):
        pl.semaphore_signal(barrier, inc=1, device_id=(nbr,),
                            device_id_type=pl.DeviceIdType.MESH)
    pl.semaphore_wait(barrier, 2)

    HH = H // 2

    def _hop(src_slot, dst_slot, hop):
        rs = []
        for buf, src_hbm, send, recv in (
                (ck, k_hbm, k_send, k_recv), (cv, v_hbm, v_send, v_recv)):
            for d, (lo, hi) in enumerate(((0, HH), (HH, H))):
                hsl = slice(lo, hi)
                src = (src_hbm.at[hsl] if src_slot is None
                       else buf.at[src_slot, hsl])
                r = pltpu.make_async_remote_copy(
                    src_ref=src, dst_ref=buf.at[dst_slot, hsl],
                    send_sem=send.at[d, hop], recv_sem=recv.at[d, hop],
                    device_id=(right,) if d == 0 else (left,),
                    device_id_type=pl.DeviceIdType.MESH)
                r.start()
                rs.append(r)
        return rs

    cp_k = pltpu.make_async_copy(k_hbm, ck.at[0], cp_sem.at[0])
    cp_v = pltpu.make_async_copy(v_hbm, cv.at[0], cp_sem.at[1])
    cp_k.start()
    cp_v.start()
    hop_rs = _hop(None, 1, 0)

    NR = QB // BLK
    qi = (lax.broadcasted_iota(jnp.int32, (NT, NR, S), 0) * NR
          + lax.broadcasted_iota(jnp.int32, (NT, NR, S), 1))
    kj = lax.broadcasted_iota(jnp.int32, (NT, NR, S), 2) // BLK
    mask_ref[...] = jnp.where(
        kj <= qi, jnp.float32(0.0), jnp.float32(NEG)
    ).astype(jnp.bfloat16)

    def q_pre(h, c):
        q_ref[h] = (_dot(x_ref[...], wq_ref[h]) * SCALE).astype(jnp.bfloat16)
        return c

    lax.fori_loop(0, H, q_pre, 0)

    cp_k.wait()
    cp_v.wait()

    for s in range(N_DEV):
        if s > 0:
            for r in hop_rs:
                r.wait()
            if s < N_DEV - 1:
                hop_rs = _hop(s, s + 1, s)
        last = s == N_DEV - 1

        def q_tile(qb, o_carry, s=s, last=last, kvlen=S):
            sl = pl.ds(qb * QB, QB)
            mb = mask_ref[qb]

            def one_head(h):
                qh = q_ref[h, sl, :]
                sc = _dot(qh, ck[s, h, :kvlen],
                          contract=((1,), (1,)))
                if s == 0:
                    sc3 = sc.reshape(QB // BLK, BLK, kvlen)
                    sc = (sc3 + mb[:, None, :kvlen]).reshape(QB, kvlen)
                p = jnp.exp(sc)
                ls = jnp.sum(p, axis=1)
                pv = _dot(p.astype(jnp.bfloat16),
                          cv[s, h, :kvlen])
                if s > 0:
                    masked = jnp.where(h < HH, my < s, my < N_DEV - s)
                    gate = jnp.where(masked, jnp.float32(0.0),
                                     jnp.float32(1.0))
                    ls = ls * gate
                    pv = pv * gate
                return ls, pv

            def h_body(hi, o_acc):
                res = [one_head(hi * 8 + k) for k in range(8)]
                for k, (ls, pv) in enumerate(res):
                    h = hi * 8 + k
                    if s == 0:
                        l_ref[qb, h] = ls
                        acc_ref[h, sl, :] = pv.astype(jnp.bfloat16)
                    elif not last:
                        l_ref[qb, h] = l_ref[qb, h] + ls
                        acc_ref[h, sl, :] = (acc_ref[h, sl, :] + pv).astype(
                            jnp.bfloat16)
                    else:
                        lt = l_ref[qb, h] + ls
                        at = acc_ref[h, sl, :] + pv
                        ctx = (at / lt[:, None]).astype(jnp.bfloat16)
                        o_acc = o_acc + _dot(ctx, wo_ref[h])
                return o_acc

            o_acc = lax.fori_loop(
                0, H // 8, h_body, jnp.zeros((QB, D), jnp.float32))
            if last:
                stage_ref[...] = o_acc.astype(jnp.bfloat16)
                cp = pltpu.make_async_copy(
                    stage_ref, out_ref.at[pl.ds(qb * QB, QB)], out_sem)
                cp.start()
                cp.wait()
            return o_carry

        if s == 0:
            for g in range(4):
                lax.fori_loop(
                    g * (NT // 4), (g + 1) * (NT // 4),
                    functools.partial(q_tile, kvlen=(g + 1) * (S // 4)), 0)
        else:
            lax.fori_loop(0, NT, q_tile, 0)


def kernel(x, Wq, K_ext, V_ext, Wo):
    xb = x[0].astype(jnp.bfloat16)
    wq = Wq.astype(jnp.bfloat16).reshape(D, H, DH).transpose(1, 0, 2)
    kt = K_ext[0].astype(jnp.bfloat16).transpose(1, 0, 2)
    vt = V_ext[0].astype(jnp.bfloat16).transpose(1, 0, 2)
    wo = Wo.astype(jnp.bfloat16).reshape(H, DH, D)

    out = pl.pallas_call(
        _body,
        out_shape=jax.ShapeDtypeStruct((S, D), jnp.bfloat16),
        in_specs=[
            pl.BlockSpec(memory_space=pltpu.VMEM),
            pl.BlockSpec(memory_space=pltpu.VMEM),
            pl.BlockSpec(memory_space=pl.ANY),
            pl.BlockSpec(memory_space=pl.ANY),
            pl.BlockSpec(memory_space=pltpu.VMEM),
        ],
        out_specs=pl.BlockSpec(memory_space=pl.ANY),
        scratch_shapes=[
            pltpu.VMEM((N_DEV, H, S, DH), jnp.bfloat16),
            pltpu.VMEM((N_DEV, H, S, DH), jnp.bfloat16),
            pltpu.VMEM((NT, QB // BLK, S), jnp.bfloat16),
            pltpu.VMEM((H, S, DH), jnp.bfloat16),
            pltpu.VMEM((NT, H, QB), jnp.float32),
            pltpu.VMEM((H, S, DH), jnp.bfloat16),
            pltpu.VMEM((QB, D), jnp.bfloat16),
            pltpu.SemaphoreType.DMA((2, N_DEV - 1)),
            pltpu.SemaphoreType.DMA((2, N_DEV - 1)),
            pltpu.SemaphoreType.DMA((2, N_DEV - 1)),
            pltpu.SemaphoreType.DMA((2, N_DEV - 1)),
            pltpu.SemaphoreType.DMA((2,)),
            pltpu.SemaphoreType.DMA,
        ],
        compiler_params=pltpu.CompilerParams(
            collective_id=0, vmem_limit_bytes=50 * 1024 * 1024
        ),
    )(xb, wq, kt, vt, wo)

    return out.astype(jnp.float32).reshape(1, S, D)
